# Initial kernel scaffold; baseline (speedup 1.0000x reference)
import functools

import jax
import jax.numpy as jnp
from jax import lax
from jax.experimental import pallas as pl

_TOP = 4
_NP = 256
_NEG = jnp.float32(-3.0e38)


def _body(n_real, x0_ref, x1_ref, pc_ref, w0_ref, w1_ref, wp_ref, out_ref):
    bB = x0_ref.shape[0]
    inv_n = jnp.float32(1.0) / jnp.float32(n_real)

    p0 = jnp.sum(x0_ref[...], axis=1) * inv_n
    p1 = jnp.sum(x1_ref[...], axis=1) * inv_n
    pp = jnp.sum(pc_ref[...], axis=1, keepdims=True) * inv_n

    a0 = p0.astype(jnp.bfloat16)
    a1 = p1.astype(jnp.bfloat16)
    logits = jnp.dot(a0, w0_ref[...].astype(jnp.bfloat16),
                     preferred_element_type=jnp.float32)
    logits = logits + jnp.dot(a1, w1_ref[...].astype(jnp.bfloat16),
                              preferred_element_type=jnp.float32)
    wp = wp_ref[0:1, :].astype(jnp.bfloat16).astype(jnp.float32)
    logits = logits + pp.astype(jnp.bfloat16).astype(jnp.float32) * wp

    j = lax.broadcasted_iota(jnp.int32, logits.shape, 1)
    logits = jnp.where((j & (_NP - 1)) < n_real, logits, _NEG)

    pc = pc_ref[...]
    cols = lax.broadcasted_iota(jnp.int32, pc.shape, 1)
    outs = []
    for t in range(_TOP):
        pol = logits[:, t * _NP:(t + 1) * _NP]
        m = jnp.max(pol, axis=1, keepdims=True)
        pj = lax.broadcasted_iota(jnp.int32, pol.shape, 1)
        idx = jnp.min(jnp.where(pol == m, pj, _NP), axis=1, keepdims=True)
        oh = cols == idx
        outs.append(jnp.sum(jnp.where(oh, pc, 0.0), axis=1, keepdims=True))
    o4 = jnp.concatenate(outs, axis=1)
    buf = jnp.zeros((bB, 128), jnp.float32)
    out_ref[...] = lax.dynamic_update_slice(buf, o4, (0, 0))


def kernel(x0, x1, pctr, W_gen):
    B, N, D = x0.shape
    T = _TOP

    Wr = W_gen.reshape(2 * D + 1, T, N + 1)[:, :, :N]
    Wf = jnp.pad(Wr, ((0, 0), (0, 0), (0, _NP - N))).reshape(2 * D + 1, T * _NP)
    W0 = Wf[:D]
    W1 = Wf[D:2 * D]
    wp8 = jnp.pad(Wf[2 * D][None, :], ((0, 7), (0, 0)))
    pctr_p = jnp.pad(pctr, ((0, 0), (0, _NP - N)))

    bB = 64
    grid = (B // bB,)
    out = pl.pallas_call(
        functools.partial(_body, N),
        grid=grid,
        in_specs=[
            pl.BlockSpec((bB, N, D), lambda i: (i, 0, 0)),
            pl.BlockSpec((bB, N, D), lambda i: (i, 0, 0)),
            pl.BlockSpec((bB, _NP), lambda i: (i, 0)),
            pl.BlockSpec((D, T * _NP), lambda i: (0, 0)),
            pl.BlockSpec((D, T * _NP), lambda i: (0, 0)),
            pl.BlockSpec((8, T * _NP), lambda i: (0, 0)),
        ],
        out_specs=pl.BlockSpec((bB, 128), lambda i: (i, 0)),
        out_shape=jax.ShapeDtypeStruct((B, 128), jnp.float32),
    )(x0, x1, pctr_p, W0, W1, wp8)
    return out[:, :T]


# speedup vs baseline: 3.1878x; 3.1878x over previous
import functools

import jax
import jax.numpy as jnp
from jax import lax
from jax.experimental import pallas as pl

_TOP = 4
_NP = 256
_NEG = -3.0e38


def _body(n_real, x0_ref, x1_ref, pc_ref, w0_ref, w1_ref, wp_ref, out_ref):
    bB = x0_ref.shape[0]
    inv_n = jnp.float32(1.0) / jnp.float32(n_real)

    p0 = jnp.sum(x0_ref[...], axis=1) * inv_n
    p1 = jnp.sum(x1_ref[...], axis=1) * inv_n
    pp = jnp.sum(pc_ref[...], axis=1, keepdims=True) * inv_n

    a0 = p0.astype(jnp.bfloat16)
    a1 = p1.astype(jnp.bfloat16)
    logits = jnp.dot(a0, w0_ref[...].astype(jnp.bfloat16),
                     preferred_element_type=jnp.float32)
    logits = logits + jnp.dot(a1, w1_ref[...].astype(jnp.bfloat16),
                              preferred_element_type=jnp.float32)
    wp = wp_ref[0:1, :].astype(jnp.bfloat16).astype(jnp.float32)
    logits = logits + pp.astype(jnp.bfloat16).astype(jnp.float32) * wp

    j = lax.broadcasted_iota(jnp.int32, logits.shape, 1)
    logits = jnp.where((j & (_NP - 1)) < n_real, logits,
                       jnp.float32(_NEG))

    pc = pc_ref[...]
    cols = lax.broadcasted_iota(jnp.int32, pc.shape, 1)
    outs = []
    for t in range(_TOP):
        pol = logits[:, t * _NP:(t + 1) * _NP]
        m = jnp.max(pol, axis=1, keepdims=True)
        pj = lax.broadcasted_iota(jnp.int32, pol.shape, 1)
        idx = jnp.min(jnp.where(pol == m, pj, _NP), axis=1, keepdims=True)
        oh = cols == idx
        outs.append(jnp.sum(jnp.where(oh, pc, 0.0), axis=1, keepdims=True))
    lanes = lax.broadcasted_iota(jnp.int32, (bB, 128), 1)
    buf = jnp.zeros((bB, 128), jnp.float32)
    for t in range(_TOP):
        buf = jnp.where(lanes == t, outs[t], buf)
    out_ref[...] = buf


def kernel(x0, x1, pctr, W_gen):
    B, N, D = x0.shape
    T = _TOP

    Wr = W_gen.reshape(2 * D + 1, T, N + 1)[:, :, :N]
    Wf = jnp.pad(Wr, ((0, 0), (0, 0), (0, _NP - N))).reshape(2 * D + 1, T * _NP)
    W0 = Wf[:D]
    W1 = Wf[D:2 * D]
    wp8 = jnp.pad(Wf[2 * D][None, :], ((0, 7), (0, 0)))
    pctr_p = jnp.pad(pctr, ((0, 0), (0, _NP - N)))

    bB = 64
    grid = (B // bB,)
    out = pl.pallas_call(
        functools.partial(_body, N),
        grid=grid,
        in_specs=[
            pl.BlockSpec((bB, N, D), lambda i: (i, 0, 0)),
            pl.BlockSpec((bB, N, D), lambda i: (i, 0, 0)),
            pl.BlockSpec((bB, _NP), lambda i: (i, 0)),
            pl.BlockSpec((D, T * _NP), lambda i: (0, 0)),
            pl.BlockSpec((D, T * _NP), lambda i: (0, 0)),
            pl.BlockSpec((8, T * _NP), lambda i: (0, 0)),
        ],
        out_specs=pl.BlockSpec((bB, 128), lambda i: (i, 0)),
        out_shape=jax.ShapeDtypeStruct((B, 128), jnp.float32),
    )(x0, x1, pctr_p, W0, W1, wp8)
    return out[:, :T]
